# baseline (speedup 1.0000x reference)
import functools

import jax
import jax.numpy as jnp
from jax import lax
from jax.experimental import pallas as pl
from jax.experimental.pallas import tpu as pltpu
from jax.experimental.pallas import tpu_sc as plsc

B, L, K = 8, 1024, 30
KP = 32
D = 128
N_ROWS = B * L
N_EDGE = B * L * KP
GATHER_WINDOW = 128


def _pv_kernel(hv_ref, w_ref, out_ref):
    out_ref[...] = jnp.dot(hv_ref[...], w_ref[...],
                           preferred_element_type=jnp.float32)


def _project_pv(h_V2, W1c):
    return pl.pallas_call(
        _pv_kernel,
        grid=(8,),
        in_specs=[
            pl.BlockSpec((N_ROWS // 8, D), lambda i: (i, 0)),
            pl.BlockSpec((D, D), lambda i: (0, 0)),
        ],
        out_specs=pl.BlockSpec((N_ROWS // 8, D), lambda i: (i, 0)),
        out_shape=jax.ShapeDtypeStruct((N_ROWS, D), jnp.float32),
    )(h_V2, W1c)


def _sc_gather(table, indices):
    mesh = plsc.VectorSubcoreMesh(core_axis_name="c", subcore_axis_name="s")

    @functools.partial(
        pl.kernel,
        out_type=jax.ShapeDtypeStruct((N_EDGE, D), table.dtype),
        mesh=mesh,
    )
    def gather_kernel(x_hbm, i_hbm, o_hbm):
        def body(i_vmem, o_vmem):
            pltpu.sync_copy(x_hbm.at[i_vmem.at[0]], o_vmem)

        pltpu.emit_pipeline(
            body,
            grid=(N_EDGE // GATHER_WINDOW,),
            in_specs=[pl.BlockSpec((1, GATHER_WINDOW), lambda i: (0, i))],
            out_specs=[pl.BlockSpec((GATHER_WINDOW, D), lambda i: (i, 0))],
            core_axis_name=("c", "s"),
            dimension_semantics=(pltpu.PARALLEL,),
        )(i_hbm, o_hbm)

    return gather_kernel(table, indices)


def _fused_body(hv_ref, he_ref, g_ref,
                w1a_ref, w1b_ref, b1_ref, w2_ref, b2_ref, w3_ref, b3_ref,
                ln1s_ref, ln1b_ref, wf1_ref, bf1_ref, wf2_ref, bf2_ref,
                ln2s_ref, ln2b_ref, out_ref):
    bf = jnp.bfloat16
    hv = hv_ref[...]
    s = jnp.dot(hv.astype(bf), w1a_ref[...].astype(bf),
                preferred_element_type=jnp.float32) + b1_ref[...]
    s32 = jnp.broadcast_to(s[:, None, :], (TILE, KP, D)).reshape(TILE * KP, D)

    he32 = jnp.pad(he_ref[0], ((0, 0), (0, KP - K), (0, 0))).reshape(TILE * KP, D)

    m1 = jnp.maximum(
        jnp.dot(he32.astype(bf), w1b_ref[...].astype(bf),
                preferred_element_type=jnp.float32) + g_ref[...] + s32, 0.0)
    m2 = jnp.maximum(
        jnp.dot(m1.astype(bf), w2_ref[...].astype(bf),
                preferred_element_type=jnp.float32) + b2_ref[...], 0.0)

    m2_3 = m2.reshape(TILE, KP, D)
    kmask = lax.broadcasted_iota(jnp.int32, (TILE, KP, D), 1) < K
    acc = jnp.sum(jnp.where(kmask, m2_3, 0.0), axis=1)

    dh = jnp.dot((acc * (1.0 / K)).astype(bf), w3_ref[...].astype(bf),
                 preferred_element_type=jnp.float32) + b3_ref[...]
    r = hv + dh
    mu = jnp.mean(r, axis=-1, keepdims=True)
    var = jnp.mean((r - mu) ** 2, axis=-1, keepdims=True)
    h = (r - mu) / jnp.sqrt(var + 1e-5) * ln1s_ref[...] + ln1b_ref[...]
    ff = jnp.maximum(
        jnp.dot(h.astype(bf), wf1_ref[...].astype(bf),
                preferred_element_type=jnp.float32) + bf1_ref[...], 0.0)
    ff = jnp.dot(ff.astype(bf), wf2_ref[...].astype(bf),
                 preferred_element_type=jnp.float32) + bf2_ref[...]
    r2 = h + ff
    mu2 = jnp.mean(r2, axis=-1, keepdims=True)
    var2 = jnp.mean((r2 - mu2) ** 2, axis=-1, keepdims=True)
    out_ref[...] = (r2 - mu2) / jnp.sqrt(var2 + 1e-5) * ln2s_ref[...] + ln2b_ref[...]


TILE = 256


def _fused_tc(h_V2, h_E, G2, W1a, W1b, b1, W2_w, b2, W3_w, b3,
              ln1_s, ln1_b, Wff1_w, bff1, Wff2_w, bff2, ln2_s, ln2_b):
    n_tiles = N_ROWS // TILE
    lpt = L // TILE
    full = lambda r, c: pl.BlockSpec((r, c), lambda i: (0, 0))
    return pl.pallas_call(
        _fused_body,
        grid=(n_tiles,),
        in_specs=[
            pl.BlockSpec((TILE, D), lambda i: (i, 0)),
            pl.BlockSpec((1, TILE, K, D),
                         lambda i: (i // lpt, i % lpt, 0, 0)),
            pl.BlockSpec((TILE * KP, D), lambda i: (i, 0)),
            full(D, D), full(D, D), full(1, D),
            full(D, D), full(1, D),
            full(D, D), full(1, D),
            full(1, D), full(1, D),
            full(D, 4 * D), full(1, 4 * D),
            full(4 * D, D), full(1, D),
            full(1, D), full(1, D),
        ],
        out_specs=pl.BlockSpec((TILE, D), lambda i: (i, 0)),
        out_shape=jax.ShapeDtypeStruct((N_ROWS, D), jnp.float32),
        compiler_params=pltpu.CompilerParams(
            dimension_semantics=("parallel",)),
    )(h_V2, h_E, G2, W1a, W1b, b1, W2_w, b2, W3_w, b3,
      ln1_s, ln1_b, Wff1_w, bff1, Wff2_w, bff2, ln2_s, ln2_b)


def kernel(h_V, h_E, E_idx, W1_w, W1_b, W2_w, W2_b, W3_w, W3_b,
           ln1_s, ln1_b, Wff1_w, Wff1_b, Wff2_w, Wff2_b, ln2_s, ln2_b):
    W1a, W1b, W1c = W1_w[0:D], W1_w[D:2 * D], W1_w[2 * D:3 * D]

    h_V2 = h_V.reshape(N_ROWS, D)

    idx = (E_idx.astype(jnp.int32)
           + (jnp.arange(B, dtype=jnp.int32) * L)[:, None, None])
    idx32 = jnp.pad(idx, ((0, 0), (0, 0), (0, KP - K)))
    idx_flat = idx32.reshape(1, N_EDGE)

    PV = _project_pv(h_V2, W1c)
    G = _sc_gather(PV, idx_flat)

    row = lambda v: v.reshape(1, -1)
    out = _fused_tc(h_V2, h_E, G, W1a, W1b, row(W1_b), W2_w, row(W2_b),
                    W3_w, row(W3_b), row(ln1_s), row(ln1_b),
                    Wff1_w, row(Wff1_b), Wff2_w, row(Wff2_b),
                    row(ln2_s), row(ln2_b))
    return out.reshape(B, L, D)


# speedup vs baseline: 3.5167x; 3.5167x over previous
import functools

import jax
import jax.numpy as jnp
from jax import lax
from jax.experimental import pallas as pl
from jax.experimental.pallas import tpu as pltpu
from jax.experimental.pallas import tpu_sc as plsc

B, L, K = 8, 1024, 30
KP = 32
D = 128
N_ROWS = B * L
N_EDGE = B * L * KP
GATHER_WINDOW = 128


def _pv_kernel(hv_ref, w_ref, out_ref):
    out_ref[...] = jnp.dot(hv_ref[...], w_ref[...],
                           preferred_element_type=jnp.float32)


def _project_pv(h_V2, W1c):
    return pl.pallas_call(
        _pv_kernel,
        grid=(8,),
        in_specs=[
            pl.BlockSpec((N_ROWS // 8, D), lambda i: (i, 0)),
            pl.BlockSpec((D, D), lambda i: (0, 0)),
        ],
        out_specs=pl.BlockSpec((N_ROWS // 8, D), lambda i: (i, 0)),
        out_shape=jax.ShapeDtypeStruct((N_ROWS, D), jnp.float32),
    )(h_V2, W1c)


def _sc_gather(table, indices):
    mesh = plsc.VectorSubcoreMesh(core_axis_name="c", subcore_axis_name="s")

    @functools.partial(
        pl.kernel,
        out_type=jax.ShapeDtypeStruct((N_EDGE, D), table.dtype),
        mesh=mesh,
    )
    def gather_kernel(x_hbm, i_hbm, o_hbm):
        def body(i_vmem, o_vmem):
            pltpu.sync_copy(x_hbm.at[i_vmem.at[0]], o_vmem)

        pltpu.emit_pipeline(
            body,
            grid=(N_EDGE // GATHER_WINDOW,),
            in_specs=[pl.BlockSpec((1, GATHER_WINDOW), lambda i: (0, i))],
            out_specs=[pl.BlockSpec((GATHER_WINDOW, D), lambda i: (i, 0))],
            core_axis_name=("c", "s"),
            dimension_semantics=(pltpu.PARALLEL,),
        )(i_hbm, o_hbm)

    return gather_kernel(table, indices)


def _fused_body(hv_ref, he_ref, g_ref,
                w1a_ref, w1b_ref, b1_ref, w2_ref, b2_ref, w3_ref, b3_ref,
                ln1s_ref, ln1b_ref, wf1_ref, bf1_ref, wf2_ref, bf2_ref,
                ln2s_ref, ln2b_ref, out_ref):
    bf = jnp.bfloat16
    hv = hv_ref[...]
    s = jnp.dot(hv.astype(bf), w1a_ref[...].astype(bf),
                preferred_element_type=jnp.float32) + b1_ref[...]
    s32 = jnp.broadcast_to(s[:, None, :], (TILE, KP, D)).reshape(TILE * KP, D)

    he32 = jnp.pad(he_ref[0], ((0, 0), (0, KP - K), (0, 0))).reshape(TILE * KP, D)

    m1 = jnp.maximum(
        jnp.dot(he32.astype(bf), w1b_ref[...].astype(bf),
                preferred_element_type=jnp.float32) + g_ref[...] + s32, 0.0)
    m2 = jnp.maximum(
        jnp.dot(m1.astype(bf), w2_ref[...].astype(bf),
                preferred_element_type=jnp.float32) + b2_ref[...], 0.0)

    m2_3 = m2.reshape(TILE, KP, D)
    kmask = lax.broadcasted_iota(jnp.int32, (TILE, KP, D), 1) < K
    acc = jnp.sum(jnp.where(kmask, m2_3, 0.0), axis=1)

    dh = jnp.dot((acc * (1.0 / K)).astype(bf), w3_ref[...].astype(bf),
                 preferred_element_type=jnp.float32) + b3_ref[...]
    r = hv + dh
    mu = jnp.mean(r, axis=-1, keepdims=True)
    var = jnp.mean((r - mu) ** 2, axis=-1, keepdims=True)
    h = (r - mu) / jnp.sqrt(var + 1e-5) * ln1s_ref[...] + ln1b_ref[...]
    ff = jnp.maximum(
        jnp.dot(h.astype(bf), wf1_ref[...].astype(bf),
                preferred_element_type=jnp.float32) + bf1_ref[...], 0.0)
    ff = jnp.dot(ff.astype(bf), wf2_ref[...].astype(bf),
                 preferred_element_type=jnp.float32) + bf2_ref[...]
    r2 = h + ff
    mu2 = jnp.mean(r2, axis=-1, keepdims=True)
    var2 = jnp.mean((r2 - mu2) ** 2, axis=-1, keepdims=True)
    out_ref[...] = (r2 - mu2) / jnp.sqrt(var2 + 1e-5) * ln2s_ref[...] + ln2b_ref[...]


TILE = 256


def _fused_tc(h_V2, h_E, G2, W1a, W1b, b1, W2_w, b2, W3_w, b3,
              ln1_s, ln1_b, Wff1_w, bff1, Wff2_w, bff2, ln2_s, ln2_b):
    n_tiles = N_ROWS // TILE
    lpt = L // TILE
    full = lambda r, c: pl.BlockSpec((r, c), lambda i: (0, 0))
    return pl.pallas_call(
        _fused_body,
        grid=(n_tiles,),
        in_specs=[
            pl.BlockSpec((TILE, D), lambda i: (i, 0)),
            pl.BlockSpec((1, TILE, K, D),
                         lambda i: (i // lpt, i % lpt, 0, 0)),
            pl.BlockSpec((TILE * KP, D), lambda i: (i, 0)),
            full(D, D), full(D, D), full(1, D),
            full(D, D), full(1, D),
            full(D, D), full(1, D),
            full(1, D), full(1, D),
            full(D, 4 * D), full(1, 4 * D),
            full(4 * D, D), full(1, D),
            full(1, D), full(1, D),
        ],
        out_specs=pl.BlockSpec((TILE, D), lambda i: (i, 0)),
        out_shape=jax.ShapeDtypeStruct((N_ROWS, D), jnp.float32),
        compiler_params=pltpu.CompilerParams(
            dimension_semantics=("parallel",)),
    )(h_V2, h_E, G2, W1a, W1b, b1, W2_w, b2, W3_w, b3,
      ln1_s, ln1_b, Wff1_w, bff1, Wff2_w, bff2, ln2_s, ln2_b)


def kernel(h_V, h_E, E_idx, W1_w, W1_b, W2_w, W2_b, W3_w, W3_b,
           ln1_s, ln1_b, Wff1_w, Wff1_b, Wff2_w, Wff2_b, ln2_s, ln2_b):
    W1a, W1b, W1c = W1_w[0:D], W1_w[D:2 * D], W1_w[2 * D:3 * D]

    h_V2 = h_V.reshape(N_ROWS, D)

    idx = (E_idx.astype(jnp.int32)
           + (jnp.arange(B, dtype=jnp.int32) * L)[:, None, None])
    own = (jnp.arange(L, dtype=jnp.int32)[None, :]
           + (jnp.arange(B, dtype=jnp.int32) * L)[:, None])
    idx32 = jnp.concatenate(
        [idx, jnp.broadcast_to(own[:, :, None], (B, L, KP - K))], axis=2)
    idx_flat = idx32.reshape(1, N_EDGE)

    PV = _project_pv(h_V2, W1c)
    G = _sc_gather(PV, idx_flat)

    row = lambda v: v.reshape(1, -1)
    out = _fused_tc(h_V2, h_E, G, W1a, W1b, row(W1_b), W2_w, row(W2_b),
                    W3_w, row(W3_b), row(ln1_s), row(ln1_b),
                    Wff1_w, row(Wff1_b), Wff2_w, row(Wff2_b),
                    row(ln2_s), row(ln2_b))
    return out.reshape(B, L, D)
